# Initial kernel scaffold; baseline (speedup 1.0000x reference)
import jax
import jax.numpy as jnp
from jax.experimental import pallas as pl


def kernel(distances, nidx):
    raise NotImplementedError("write your pallas kernel here")


# speedup vs baseline: 1.7968x; 1.7968x over previous
import functools

import jax
import jax.numpy as jnp
from jax import lax
from jax.experimental import pallas as pl
from jax.experimental.pallas import tpu as pltpu
from jax.experimental.pallas import tpu_sc as plsc

N = 100000
M = 64
K = 32
G = 16
NGROUPS = N // G
NC = 2
NS = 16
NW = NC * NS
T = (NGROUPS + NW - 1) // NW


def _batcher(n):
    comps = []

    def merge(lo, n_, r):
        m = r * 2
        if m < n_:
            merge(lo, n_, m)
            merge(lo + r, n_, m)
            for i in range(lo + r, lo + n_ - r, m):
                comps.append((i, i + r))
        else:
            comps.append((lo, lo + r))

    def sort(lo, n_):
        if n_ > 1:
            m = n_ // 2
            sort(lo, m)
            sort(lo + m, m)
            merge(lo, n_, 1)

    sort(0, n)
    return comps


def _network():
    comps = [c for c in _batcher(M) if c[0] != 0]
    needed = set(range(1, K))
    kept = []
    for (i, j) in reversed(comps):
        if i in needed or j in needed:
            kept.append((i, j))
            needed.add(i)
            needed.add(j)
    kept.reverse()
    return kept


_COMPS = _network()


def _sc_sort(dist_hbm, nidx_hbm, outd_hbm, outn_hbm, dist_v, nidx_v, outd_v, outn_v):
    wid = lax.axis_index("s") * NC + lax.axis_index("c")
    rows = lax.iota(jnp.int32, G)

    def body(t, carry):
        g = wid + NW * t

        @pl.when(g < NGROUPS)
        def _():
            base = g * G
            pltpu.sync_copy(dist_hbm.at[pl.ds(base, G)], dist_v)
            pltpu.sync_copy(nidx_hbm.at[pl.ds(base, G)], nidx_v)

            keys = [None] * M
            cols = [None] * M
            for j in range(1, M):
                cj = jnp.full((G,), j, jnp.int32)
                keys[j] = plsc.load_gather(dist_v, [rows, cj])
                cols[j] = cj

            for (i, j) in _COMPS:
                ka, kb = keys[i], keys[j]
                pa, pb = cols[i], cols[j]
                swap = kb < ka
                keys[i] = jnp.minimum(ka, kb)
                keys[j] = jnp.maximum(ka, kb)
                cols[i] = jnp.where(swap, pb, pa)
                cols[j] = jnp.where(swap, pa, pb)

            c0 = jnp.full((G,), 0, jnp.int32)
            plsc.store_scatter(outd_v, [rows, c0],
                               plsc.load_gather(dist_v, [rows, c0]))
            plsc.store_scatter(outn_v, [rows, c0],
                               plsc.load_gather(nidx_v, [rows, c0]))
            for p in range(1, K):
                cp = jnp.full((G,), p, jnp.int32)
                plsc.store_scatter(outd_v, [rows, cp], keys[p])
                plsc.store_scatter(outn_v, [rows, cp],
                                   plsc.load_gather(nidx_v, [rows, cols[p]]))

            pltpu.sync_copy(outd_v, outd_hbm.at[pl.ds(base, G)])
            pltpu.sync_copy(outn_v, outn_hbm.at[pl.ds(base, G)])

        return carry

    lax.fori_loop(0, T, body, 0)


@jax.jit
def kernel(distances, nidx):
    run = functools.partial(
        pl.kernel,
        out_type=(jax.ShapeDtypeStruct((N, K), jnp.float32),
                  jax.ShapeDtypeStruct((N, K), jnp.int32)),
        mesh=plsc.VectorSubcoreMesh(core_axis_name="c", subcore_axis_name="s"),
        compiler_params=pltpu.CompilerParams(needs_layout_passes=False),
        scratch_types=[
            pltpu.VMEM((G, M), jnp.float32),
            pltpu.VMEM((G, M), jnp.int32),
            pltpu.VMEM((G, K), jnp.float32),
            pltpu.VMEM((G, K), jnp.int32),
        ],
    )(_sc_sort)
    return run(distances, nidx)


# speedup vs baseline: 2.6585x; 1.4796x over previous
import functools

import jax
import jax.numpy as jnp
from jax import lax
from jax.experimental import pallas as pl
from jax.experimental.pallas import tpu as pltpu
from jax.experimental.pallas import tpu_sc as plsc

N = 100000
M = 64
K = 32
G = 16
NGROUPS = N // G
NC = 2
NS = 16
NW = NC * NS
T = (NGROUPS + NW - 1) // NW


def _batcher(n):
    comps = []

    def merge(lo, n_, r):
        m = r * 2
        if m < n_:
            merge(lo, n_, m)
            merge(lo + r, n_, m)
            for i in range(lo + r, lo + n_ - r, m):
                comps.append((i, i + r))
        else:
            comps.append((lo, lo + r))

    def sort(lo, n_):
        if n_ > 1:
            m = n_ // 2
            sort(lo, m)
            sort(lo + m, m)
            merge(lo, n_, 1)

    sort(0, n)
    return comps


def _network():
    comps = [c for c in _batcher(M) if c[0] != 0]
    needed = set(range(1, K))
    kept = []
    for (i, j) in reversed(comps):
        if i in needed or j in needed:
            kept.append((i, j))
            needed.add(i)
            needed.add(j)
    kept.reverse()
    return kept


_COMPS = _network()


def _sc_sort(dist_hbm, nidx_hbm, outd_hbm, outn_hbm,
             dist_v0, dist_v1, nidx_v0, nidx_v1,
             outd_v0, outd_v1, outn_v0, outn_v1,
             ind_s0, ind_s1, inn_s0, inn_s1,
             outd_s0, outd_s1, outn_s0, outn_s1):
    wid = lax.axis_index("s") * NC + lax.axis_index("c")
    rows = lax.iota(jnp.int32, G)
    dist_v = (dist_v0, dist_v1)
    nidx_v = (nidx_v0, nidx_v1)
    outd_v = (outd_v0, outd_v1)
    outn_v = (outn_v0, outn_v1)
    ind_s = (ind_s0, ind_s1)
    inn_s = (inn_s0, inn_s1)
    outd_s = (outd_s0, outd_s1)
    outn_s = (outn_s0, outn_s1)

    def fetch(t, b):
        base = (wid + NW * t) * G
        pltpu.make_async_copy(
            dist_hbm.at[pl.ds(base, G)], dist_v[b], ind_s[b]).start()
        pltpu.make_async_copy(
            nidx_hbm.at[pl.ds(base, G)], nidx_v[b], inn_s[b]).start()

    fetch(0, 0)
    fetch(1, 1)

    def body(tt, carry):
        for b in range(2):
            t = 2 * tt + b
            g = wid + NW * t

            @pl.when(g < NGROUPS)
            def _():
                base = g * G
                pltpu.make_async_copy(
                    dist_hbm.at[pl.ds(base, G)], dist_v[b], ind_s[b]).wait()
                pltpu.make_async_copy(
                    nidx_hbm.at[pl.ds(base, G)], nidx_v[b], inn_s[b]).wait()

                keys = [None] * M
                cols = [None] * M
                for j in range(1, M):
                    cj = jnp.full((G,), j, jnp.int32)
                    keys[j] = plsc.load_gather(dist_v[b], [rows, cj])
                    cols[j] = cj

                for (i, j) in _COMPS:
                    ka, kb = keys[i], keys[j]
                    pa, pb = cols[i], cols[j]
                    swap = kb < ka
                    keys[i] = jnp.minimum(ka, kb)
                    keys[j] = jnp.maximum(ka, kb)
                    cols[i] = jnp.where(swap, pb, pa)
                    cols[j] = jnp.where(swap, pa, pb)

                @pl.when(t >= 2)
                def _():
                    pltpu.make_async_copy(
                        outd_v[b], outd_hbm.at[pl.ds(base, G)],
                        outd_s[b]).wait()
                    pltpu.make_async_copy(
                        outn_v[b], outn_hbm.at[pl.ds(base, G)],
                        outn_s[b]).wait()

                c0 = jnp.full((G,), 0, jnp.int32)
                plsc.store_scatter(outd_v[b], [rows, c0],
                                   plsc.load_gather(dist_v[b], [rows, c0]))
                plsc.store_scatter(outn_v[b], [rows, c0],
                                   plsc.load_gather(nidx_v[b], [rows, c0]))
                for p in range(1, K):
                    cp = jnp.full((G,), p, jnp.int32)
                    plsc.store_scatter(outd_v[b], [rows, cp], keys[p])
                    plsc.store_scatter(outn_v[b], [rows, cp],
                                       plsc.load_gather(nidx_v[b], [rows, cols[p]]))

                pltpu.make_async_copy(
                    outd_v[b], outd_hbm.at[pl.ds(base, G)], outd_s[b]).start()
                pltpu.make_async_copy(
                    outn_v[b], outn_hbm.at[pl.ds(base, G)], outn_s[b]).start()

                @pl.when(g + 2 * NW < NGROUPS)
                def _():
                    fetch(t + 2, b)

        return carry

    lax.fori_loop(0, T // 2, body, 0)

    pltpu.make_async_copy(
        outd_v[0], outd_hbm.at[pl.ds(0, G)], outd_s[0]).wait()
    pltpu.make_async_copy(
        outn_v[0], outn_hbm.at[pl.ds(0, G)], outn_s[0]).wait()

    @pl.when(wid + NW * (T - 1) < NGROUPS)
    def _():
        pltpu.make_async_copy(
            outd_v[1], outd_hbm.at[pl.ds(0, G)], outd_s[1]).wait()
        pltpu.make_async_copy(
            outn_v[1], outn_hbm.at[pl.ds(0, G)], outn_s[1]).wait()


@jax.jit
def kernel(distances, nidx):
    run = functools.partial(
        pl.kernel,
        out_type=(jax.ShapeDtypeStruct((N, K), jnp.float32),
                  jax.ShapeDtypeStruct((N, K), jnp.int32)),
        mesh=plsc.VectorSubcoreMesh(core_axis_name="c", subcore_axis_name="s"),
        compiler_params=pltpu.CompilerParams(needs_layout_passes=False),
        scratch_types=[
            pltpu.VMEM((G, M), jnp.float32),
            pltpu.VMEM((G, M), jnp.float32),
            pltpu.VMEM((G, M), jnp.int32),
            pltpu.VMEM((G, M), jnp.int32),
            pltpu.VMEM((G, K), jnp.float32),
            pltpu.VMEM((G, K), jnp.float32),
            pltpu.VMEM((G, K), jnp.int32),
            pltpu.VMEM((G, K), jnp.int32),
        ] + [pltpu.SemaphoreType.DMA] * 8,
    )(_sc_sort)
    return run(distances, nidx)
